# baseline (speedup 1.0000x reference)
import jax
import jax.numpy as jnp
from jax import lax
from jax.experimental import pallas as pl
from jax.experimental.pallas import tpu as pltpu
from jax.experimental.pallas import tpu_sc as plsc

N_NODES = 10000
N_EDGES = 160000
N_REL = 4
IN_DIM = 532
OUT_DIM = 512
OC = 4
LANES = 128
NC = 2
NS = 16
NW = NC * NS
EPW = N_EDGES // NW
BATCH = 128
NB = 40
PAD = NB * BATCH - EPW
TBL = (N_REL + 1) * N_NODES
AGG_ROWS = 10112
ZSTRIPE = AGG_ROWS // NS
BN = 400


def _proj_body(x_ref, w_ref, o_ref):
    r = pl.program_id(1)
    oc = pl.program_id(2)
    o_ref[...] = jnp.dot(x_ref[...], w_ref[r * OC + oc],
                         preferred_element_type=jnp.float32)


def _combine_body(hrs_ref, agg_ref, deg_ref, b_ref, o_ref):
    a = agg_ref[0, 0] + agg_ref[1, 0]
    dg = deg_ref[0, :, :1] + deg_ref[1, :, :1]
    norm = 1.0 / jnp.maximum(dg, 1.0)
    o_ref[...] = jnp.maximum(hrs_ref[...] + a * norm + b_ref[0], 0.0)


def _sc_agg_body(hr, ridxP, dstP, aggp, degp, idxv, dstv, rows, zbuf,
                 aggsp, semA, semB):
    cid = lax.axis_index("c")
    sid = lax.axis_index("s")
    w = cid * NS + sid

    pltpu.sync_copy(ridxP.at[w], idxv)
    pltpu.sync_copy(dstP.at[w], dstv)

    def _fill_zbuf(i, _):
        for k in range(LANES // 16):
            zbuf[i, pl.ds(k * 16, 16)] = jnp.zeros((16,), jnp.float32)
        return 0
    lax.fori_loop(0, 32, _fill_zbuf, 0)

    zrow = sid * ZSTRIPE

    for oc in range(OC):
        if oc > 0:
            def _bump_body(j, _):
                for k in range(BATCH // 16):
                    sl = pl.ds(k * 16, 16)
                    idxv[j, sl] = idxv[j, sl] + TBL
                return 0
            lax.fori_loop(0, NB, _bump_body, 0)

        for z in range(19):
            pltpu.sync_copy(zbuf, aggsp.at[pl.ds(zrow + z * 32, 32)])
        pltpu.sync_copy(zbuf.at[pl.ds(0, 24)], aggsp.at[pl.ds(zrow + 608, 24)])
        plsc.subcore_barrier()

        pltpu.async_copy(hr.at[idxv.at[0]], rows.at[0], semA)

        def _pair(p, _):
            j0 = 2 * p
            j1 = 2 * p + 1
            j2 = jnp.where(j1 + 1 < NB, j1 + 1, 0)
            pltpu.async_copy(hr.at[idxv.at[j1]], rows.at[1], semB)
            pltpu.make_async_copy(hr.at[idxv.at[j0]], rows.at[0], semA).wait()
            pltpu.sync_copy(rows.at[0], aggsp.at[dstv.at[j0]], add=True)
            pltpu.async_copy(hr.at[idxv.at[j2]], rows.at[0], semA)
            pltpu.make_async_copy(hr.at[idxv.at[j1]], rows.at[1], semB).wait()
            pltpu.sync_copy(rows.at[1], aggsp.at[dstv.at[j1]], add=True)
            return 0
        lax.fori_loop(0, NB // 2, _pair, 0)
        pltpu.make_async_copy(hr.at[idxv.at[0]], rows.at[0], semA).wait()
        plsc.subcore_barrier()

        for qo, qn in ((0, 160), (160, 160), (320, 160), (480, 152)):
            pltpu.sync_copy(aggsp.at[pl.ds(zrow + qo, qn)],
                            aggp.at[cid, oc, pl.ds(zrow + qo, qn)])
        plsc.subcore_barrier()

    def _fill_ones(i, _):
        for k in range(LANES // 16):
            rows[0, i, pl.ds(k * 16, 16)] = jnp.ones((16,), jnp.float32)
        return 0
    lax.fori_loop(0, BATCH, _fill_ones, 0)
    for z in range(19):
        pltpu.sync_copy(zbuf, aggsp.at[pl.ds(zrow + z * 32, 32)])
    pltpu.sync_copy(zbuf.at[pl.ds(0, 24)], aggsp.at[pl.ds(zrow + 608, 24)])
    plsc.subcore_barrier()

    def _deg_body(j, _):
        pltpu.sync_copy(rows.at[0], aggsp.at[dstv.at[j]], add=True)
        return 0
    lax.fori_loop(0, NB, _deg_body, 0)
    plsc.subcore_barrier()
    for qo, qn in ((0, 160), (160, 160), (320, 160), (480, 152)):
        pltpu.sync_copy(aggsp.at[pl.ds(zrow + qo, qn)],
                        degp.at[cid, pl.ds(zrow + qo, qn)])


def kernel(x, edge_index, edge_type, W_rel, W_self, b):
    src = edge_index[0]
    dst = edge_index[1]

    W_all = jnp.concatenate([W_rel, W_self[None]], axis=0)
    W20 = W_all.reshape(N_REL + 1, IN_DIM, OC, LANES)
    W20 = W20.transpose(0, 2, 1, 3).reshape((N_REL + 1) * OC, IN_DIM, LANES)
    W20 = W20.astype(jnp.bfloat16)
    xb = x.astype(jnp.bfloat16)

    ridx = edge_type * N_NODES + src
    wi = jnp.arange(NW, dtype=jnp.int32)[:, None]
    ki = jnp.arange(PAD, dtype=jnp.int32)[None, :]
    pad_ridx = N_REL * N_NODES + (wi * PAD + ki) % N_NODES
    pad_dst = N_NODES + (wi % NS) + jnp.zeros_like(ki)
    ridxP = jnp.concatenate([ridx.reshape(NW, EPW), pad_ridx], 1)
    ridxP = ridxP.reshape(NW, NB, BATCH)
    dstP = jnp.concatenate([dst.reshape(NW, EPW), pad_dst], 1)
    dstP = dstP.reshape(NW, NB, BATCH)

    hr = pl.pallas_call(
        _proj_body,
        grid=(N_NODES // BN, N_REL + 1, OC),
        in_specs=[
            pl.BlockSpec((BN, IN_DIM), lambda i, r, oc: (i, 0)),
            pl.BlockSpec(((N_REL + 1) * OC, IN_DIM, LANES),
                         lambda i, r, oc: (0, 0, 0)),
        ],
        out_specs=pl.BlockSpec(
            (BN, LANES),
            lambda i, r, oc: (oc * (TBL // BN) + r * (N_NODES // BN) + i, 0)),
        out_shape=jax.ShapeDtypeStruct((OC * TBL, LANES), jnp.float32),
    )(xb, W20)

    mesh = plsc.VectorSubcoreMesh(core_axis_name="c", subcore_axis_name="s")
    sc_agg = pl.kernel(
        _sc_agg_body,
        out_type=[
            jax.ShapeDtypeStruct((NC, OC, AGG_ROWS, LANES), jnp.float32),
            jax.ShapeDtypeStruct((NC, AGG_ROWS, LANES), jnp.float32),
        ],
        mesh=mesh,
        scratch_types=[
            pltpu.VMEM((NB, BATCH), jnp.int32),
            pltpu.VMEM((NB, BATCH), jnp.int32),
            pltpu.VMEM((2, BATCH, LANES), jnp.float32),
            pltpu.VMEM((32, LANES), jnp.float32),
            pltpu.VMEM_SHARED((AGG_ROWS, LANES), jnp.float32),
            pltpu.SemaphoreType.DMA,
            pltpu.SemaphoreType.DMA,
        ],
    )
    aggp, degp = sc_agg(hr, ridxP, dstP)

    b4 = b.reshape(OC, 1, LANES)

    out = pl.pallas_call(
        _combine_body,
        grid=(N_NODES // BN, OC),
        in_specs=[
            pl.BlockSpec((BN, LANES),
                         lambda i, oc: (oc * (TBL // BN) + N_REL * (N_NODES // BN) + i, 0)),
            pl.BlockSpec((NC, 1, BN, LANES), lambda i, oc: (0, oc, i, 0)),
            pl.BlockSpec((NC, BN, LANES), lambda i, oc: (0, i, 0)),
            pl.BlockSpec((1, 1, LANES), lambda i, oc: (oc, 0, 0)),
        ],
        out_specs=pl.BlockSpec((BN, LANES), lambda i, oc: (i, oc)),
        out_shape=jax.ShapeDtypeStruct((N_NODES, OUT_DIM), jnp.float32),
    )(hr, aggp, degp, b4)
    return out


# speedup vs baseline: 1.0929x; 1.0929x over previous
import jax
import jax.numpy as jnp
from jax import lax
from jax.experimental import pallas as pl
from jax.experimental.pallas import tpu as pltpu
from jax.experimental.pallas import tpu_sc as plsc

N_NODES = 10000
N_EDGES = 160000
N_REL = 4
IN_DIM = 532
OUT_DIM = 512
OC = 4
LANES = 128
NC = 2
NS = 16
NW = NC * NS
EPW = N_EDGES // NW
BATCH = 128
NB = 40
PAD = NB * BATCH - EPW
TBL = N_REL * N_NODES
AGG_ROWS = 10112
ZSTRIPE = AGG_ROWS // NS
BN = 400


def _proj_body(x_ref, w_ref, o_ref):
    r = pl.program_id(1)
    oc = pl.program_id(2)
    o_ref[...] = jnp.dot(x_ref[...], w_ref[r * OC + oc],
                         preferred_element_type=jnp.float32)


def _combine_body(x_ref, ws_ref, agg_ref, deg_ref, b_ref, o_ref):
    self_term = jnp.dot(x_ref[...], ws_ref[...],
                        preferred_element_type=jnp.float32)
    a = agg_ref[0, 0] + agg_ref[1, 0]
    dg = deg_ref[0, :, :1] + deg_ref[1, :, :1]
    norm = 1.0 / jnp.maximum(dg, 1.0)
    o_ref[...] = jnp.maximum(self_term + a * norm + b_ref[0], 0.0)


def _sc_agg_body(hr, ridxP, dstP, aggp, degp, idxv, dstv, rows, zbuf,
                 aggsp, semA, semB):
    cid = lax.axis_index("c")
    sid = lax.axis_index("s")
    w = cid * NS + sid

    pltpu.sync_copy(ridxP.at[w], idxv)
    pltpu.sync_copy(dstP.at[w], dstv)

    def _fill_zbuf(i, _):
        for k in range(LANES // 16):
            zbuf[i, pl.ds(k * 16, 16)] = jnp.zeros((16,), jnp.float32)
        return 0
    lax.fori_loop(0, 32, _fill_zbuf, 0)

    zrow = sid * ZSTRIPE

    for oc in range(OC):
        if oc > 0:
            def _bump_body(j, _):
                for k in range(BATCH // 16):
                    sl = pl.ds(k * 16, 16)
                    idxv[j, sl] = idxv[j, sl] + TBL
                return 0
            lax.fori_loop(0, NB, _bump_body, 0)

        for z in range(19):
            pltpu.sync_copy(zbuf, aggsp.at[pl.ds(zrow + z * 32, 32)])
        pltpu.sync_copy(zbuf.at[pl.ds(0, 24)], aggsp.at[pl.ds(zrow + 608, 24)])
        plsc.subcore_barrier()

        pltpu.async_copy(hr.at[idxv.at[0]], rows.at[0], semA)

        def _pair(p, _):
            j0 = 2 * p
            j1 = 2 * p + 1
            j2 = jnp.where(j1 + 1 < NB, j1 + 1, 0)
            pltpu.async_copy(hr.at[idxv.at[j1]], rows.at[1], semB)
            pltpu.make_async_copy(hr.at[idxv.at[j0]], rows.at[0], semA).wait()
            pltpu.sync_copy(rows.at[0], aggsp.at[dstv.at[j0]], add=True)
            pltpu.async_copy(hr.at[idxv.at[j2]], rows.at[0], semA)
            pltpu.make_async_copy(hr.at[idxv.at[j1]], rows.at[1], semB).wait()
            pltpu.sync_copy(rows.at[1], aggsp.at[dstv.at[j1]], add=True)
            return 0
        lax.fori_loop(0, NB // 2, _pair, 0)
        pltpu.make_async_copy(hr.at[idxv.at[0]], rows.at[0], semA).wait()
        plsc.subcore_barrier()

        for qo, qn in ((0, 160), (160, 160), (320, 160), (480, 152)):
            pltpu.sync_copy(aggsp.at[pl.ds(zrow + qo, qn)],
                            aggp.at[cid, oc, pl.ds(zrow + qo, qn)])
        plsc.subcore_barrier()

    def _fill_ones(i, _):
        for k in range(LANES // 16):
            rows[0, i, pl.ds(k * 16, 16)] = jnp.ones((16,), jnp.float32)
        return 0
    lax.fori_loop(0, BATCH, _fill_ones, 0)
    for z in range(19):
        pltpu.sync_copy(zbuf, aggsp.at[pl.ds(zrow + z * 32, 32)])
    pltpu.sync_copy(zbuf.at[pl.ds(0, 24)], aggsp.at[pl.ds(zrow + 608, 24)])
    plsc.subcore_barrier()

    def _deg_body(j, _):
        pltpu.sync_copy(rows.at[0], aggsp.at[dstv.at[j]], add=True)
        return 0
    lax.fori_loop(0, NB, _deg_body, 0)
    plsc.subcore_barrier()
    for qo, qn in ((0, 160), (160, 160), (320, 160), (480, 152)):
        pltpu.sync_copy(aggsp.at[pl.ds(zrow + qo, qn)],
                        degp.at[cid, pl.ds(zrow + qo, qn)])


def kernel(x, edge_index, edge_type, W_rel, W_self, b):
    src = edge_index[0]
    dst = edge_index[1]

    W16 = W_rel.reshape(N_REL, IN_DIM, OC, LANES)
    W16 = W16.transpose(0, 2, 1, 3).reshape(N_REL * OC, IN_DIM, LANES)

    ridx = edge_type * N_NODES + src
    wi = jnp.arange(NW, dtype=jnp.int32)[:, None]
    ki = jnp.arange(PAD, dtype=jnp.int32)[None, :]
    pad_ridx = (wi * PAD + ki) % N_NODES
    pad_dst = N_NODES + (wi % NS) + jnp.zeros_like(ki)
    ridxP = jnp.concatenate([ridx.reshape(NW, EPW), pad_ridx], 1)
    ridxP = ridxP.reshape(NW, NB, BATCH)
    dstP = jnp.concatenate([dst.reshape(NW, EPW), pad_dst], 1)
    dstP = dstP.reshape(NW, NB, BATCH)

    hr = pl.pallas_call(
        _proj_body,
        grid=(N_NODES // BN, N_REL, OC),
        in_specs=[
            pl.BlockSpec((BN, IN_DIM), lambda i, r, oc: (i, 0)),
            pl.BlockSpec((N_REL * OC, IN_DIM, LANES),
                         lambda i, r, oc: (0, 0, 0)),
        ],
        out_specs=pl.BlockSpec(
            (BN, LANES),
            lambda i, r, oc: (oc * (TBL // BN) + r * (N_NODES // BN) + i, 0)),
        out_shape=jax.ShapeDtypeStruct((OC * TBL, LANES), jnp.float32),
    )(x, W16)

    mesh = plsc.VectorSubcoreMesh(core_axis_name="c", subcore_axis_name="s")
    sc_agg = pl.kernel(
        _sc_agg_body,
        out_type=[
            jax.ShapeDtypeStruct((NC, OC, AGG_ROWS, LANES), jnp.float32),
            jax.ShapeDtypeStruct((NC, AGG_ROWS, LANES), jnp.float32),
        ],
        mesh=mesh,
        scratch_types=[
            pltpu.VMEM((NB, BATCH), jnp.int32),
            pltpu.VMEM((NB, BATCH), jnp.int32),
            pltpu.VMEM((2, BATCH, LANES), jnp.float32),
            pltpu.VMEM((32, LANES), jnp.float32),
            pltpu.VMEM_SHARED((AGG_ROWS, LANES), jnp.float32),
            pltpu.SemaphoreType.DMA,
            pltpu.SemaphoreType.DMA,
        ],
    )
    aggp, degp = sc_agg(hr, ridxP, dstP)

    b4 = b.reshape(OC, 1, LANES)

    out = pl.pallas_call(
        _combine_body,
        grid=(N_NODES // BN, OC),
        in_specs=[
            pl.BlockSpec((BN, IN_DIM), lambda i, oc: (i, 0)),
            pl.BlockSpec((IN_DIM, LANES), lambda i, oc: (0, oc)),
            pl.BlockSpec((NC, 1, BN, LANES), lambda i, oc: (0, oc, i, 0)),
            pl.BlockSpec((NC, BN, LANES), lambda i, oc: (0, i, 0)),
            pl.BlockSpec((1, 1, LANES), lambda i, oc: (oc, 0, 0)),
        ],
        out_specs=pl.BlockSpec((BN, LANES), lambda i, oc: (i, oc)),
        out_shape=jax.ShapeDtypeStruct((N_NODES, OUT_DIM), jnp.float32),
    )(x, W_self, aggp, degp, b4)
    return out


# speedup vs baseline: 1.2994x; 1.1889x over previous
import jax
import jax.numpy as jnp
from jax import lax
from jax.experimental import pallas as pl
from jax.experimental.pallas import tpu as pltpu
from jax.experimental.pallas import tpu_sc as plsc

N_NODES = 10000
N_EDGES = 160000
N_REL = 4
IN_DIM = 532
OUT_DIM = 512
OC = 4
LANES = 128
NC = 2
NS = 16
NW = NC * NS
EPW = N_EDGES // NW
BATCH = 128
NB = 40
PAD = NB * BATCH - EPW
TBL = N_REL * N_NODES
AGG_ROWS = 10112
ZSTRIPE = AGG_ROWS // NS
BN = 400


def _proj_body(x_ref, w_ref, o_ref):
    r = pl.program_id(1)
    o_ref[...] = jnp.dot(x_ref[...], w_ref[r],
                         preferred_element_type=jnp.float32)


def _combine_body(x_ref, ws_ref, agg_ref, deg_ref, b_ref, o_ref):
    self_term = jnp.dot(x_ref[...], ws_ref[...],
                        preferred_element_type=jnp.float32)
    a = agg_ref[0, 0] + agg_ref[1, 0]
    dg = deg_ref[0, :, :1] + deg_ref[1, :, :1]
    norm = 1.0 / jnp.maximum(dg, 1.0)
    o_ref[...] = jnp.maximum(self_term + a * norm + b_ref[0], 0.0)


def _zero_stripe(zbuf, table, zrow):
    for z in range(19):
        pltpu.sync_copy(zbuf, table.at[pl.ds(zrow + z * 32, 32)])
    pltpu.sync_copy(zbuf.at[pl.ds(0, 24)], table.at[pl.ds(zrow + 608, 24)])


def _copyout_stripe(table, out, zrow):
    for qo, qn in ((0, 160), (160, 160), (320, 160), (480, 152)):
        pltpu.sync_copy(table.at[pl.ds(zrow + qo, qn)],
                        out.at[pl.ds(zrow + qo, qn)])


def _fill_zeros(buf, nrows):
    def _body(i, _):
        for k in range(LANES // 16):
            buf[i, pl.ds(k * 16, 16)] = jnp.zeros((16,), jnp.float32)
        return 0
    lax.fori_loop(0, nrows, _body, 0)


def _sc_agg_body(hr, ridxP, dstP, aggp, idxv, dstv, rows, zbuf,
                 aggsp, semA, semB):
    cid = lax.axis_index("c")
    sid = lax.axis_index("s")
    w = cid * NS + sid

    pltpu.sync_copy(ridxP.at[w], idxv)
    pltpu.sync_copy(dstP.at[w], dstv)
    _fill_zeros(zbuf, 32)

    zrow = sid * ZSTRIPE
    _zero_stripe(zbuf, aggsp, zrow)
    plsc.subcore_barrier()

    pltpu.async_copy(hr.at[idxv.at[0]], rows.at[0], semA)

    def _pair(p, _):
        j0 = 2 * p
        j1 = 2 * p + 1
        j2 = jnp.where(j1 + 1 < NB, j1 + 1, 0)
        pltpu.async_copy(hr.at[idxv.at[j1]], rows.at[1], semB)
        pltpu.make_async_copy(hr.at[idxv.at[j0]], rows.at[0], semA).wait()
        pltpu.sync_copy(rows.at[0], aggsp.at[dstv.at[j0]], add=True)
        pltpu.async_copy(hr.at[idxv.at[j2]], rows.at[0], semA)
        pltpu.make_async_copy(hr.at[idxv.at[j1]], rows.at[1], semB).wait()
        pltpu.sync_copy(rows.at[1], aggsp.at[dstv.at[j1]], add=True)
        return 0
    lax.fori_loop(0, NB // 2, _pair, 0)
    pltpu.make_async_copy(hr.at[idxv.at[0]], rows.at[0], semA).wait()
    plsc.subcore_barrier()

    _copyout_stripe(aggsp, aggp.at[cid], zrow)


def _sc_deg_body(dstP, degp, dstv, ones, zbuf, degsp):
    cid = lax.axis_index("c")
    sid = lax.axis_index("s")
    w = cid * NS + sid

    pltpu.sync_copy(dstP.at[w], dstv)
    _fill_zeros(zbuf, 32)

    def _fill_ones(i, _):
        for k in range(LANES // 16):
            ones[i, pl.ds(k * 16, 16)] = jnp.ones((16,), jnp.float32)
        return 0
    lax.fori_loop(0, BATCH, _fill_ones, 0)

    zrow = sid * ZSTRIPE
    _zero_stripe(zbuf, degsp, zrow)
    plsc.subcore_barrier()

    def _deg(j, _):
        pltpu.sync_copy(ones, degsp.at[dstv.at[j]], add=True)
        return 0
    lax.fori_loop(0, NB, _deg, 0)
    plsc.subcore_barrier()

    _copyout_stripe(degsp, degp.at[cid], zrow)


def kernel(x, edge_index, edge_type, W_rel, W_self, b):
    src = edge_index[0]
    dst = edge_index[1]

    W16 = W_rel.reshape(N_REL, IN_DIM, OC, LANES)
    W16 = W16.transpose(0, 2, 1, 3).reshape(N_REL * OC, IN_DIM, LANES)

    ridx = edge_type * N_NODES + src
    wi = jnp.arange(NW, dtype=jnp.int32)[:, None]
    ki = jnp.arange(PAD, dtype=jnp.int32)[None, :]
    pad_ridx = (wi * PAD + ki) % N_NODES
    pad_dst = N_NODES + (wi % NS) + jnp.zeros_like(ki)
    ridxP = jnp.concatenate([ridx.reshape(NW, EPW), pad_ridx], 1)
    ridxP = ridxP.reshape(NW, NB, BATCH)
    dstP = jnp.concatenate([dst.reshape(NW, EPW), pad_dst], 1)
    dstP = dstP.reshape(NW, NB, BATCH)

    mesh = plsc.VectorSubcoreMesh(core_axis_name="c", subcore_axis_name="s")

    sc_deg = pl.kernel(
        _sc_deg_body,
        out_type=[jax.ShapeDtypeStruct((NC, AGG_ROWS, LANES), jnp.float32)],
        mesh=mesh,
        scratch_types=[
            pltpu.VMEM((NB, BATCH), jnp.int32),
            pltpu.VMEM((BATCH, LANES), jnp.float32),
            pltpu.VMEM((32, LANES), jnp.float32),
            pltpu.VMEM_SHARED((AGG_ROWS, LANES), jnp.float32),
        ],
    )
    (degp,) = sc_deg(dstP)

    sc_agg = pl.kernel(
        _sc_agg_body,
        out_type=[jax.ShapeDtypeStruct((NC, AGG_ROWS, LANES), jnp.float32)],
        mesh=mesh,
        scratch_types=[
            pltpu.VMEM((NB, BATCH), jnp.int32),
            pltpu.VMEM((NB, BATCH), jnp.int32),
            pltpu.VMEM((2, BATCH, LANES), jnp.float32),
            pltpu.VMEM((32, LANES), jnp.float32),
            pltpu.VMEM_SHARED((AGG_ROWS, LANES), jnp.float32),
            pltpu.SemaphoreType.DMA,
            pltpu.SemaphoreType.DMA,
        ],
    )

    aggs = []
    for oc in range(OC):
        hr_oc = pl.pallas_call(
            _proj_body,
            grid=(N_NODES // BN, N_REL),
            in_specs=[
                pl.BlockSpec((BN, IN_DIM), lambda i, r: (i, 0)),
                pl.BlockSpec((N_REL, IN_DIM, LANES),
                             lambda i, r, _oc=oc: (0, 0, 0)),
            ],
            out_specs=pl.BlockSpec(
                (BN, LANES),
                lambda i, r: (r * (N_NODES // BN) + i, 0)),
            out_shape=jax.ShapeDtypeStruct((TBL, LANES), jnp.float32),
        )(x, lax.slice_in_dim(W16.reshape(N_REL, OC, IN_DIM, LANES),
                              oc, oc + 1, axis=1).reshape(N_REL, IN_DIM, LANES))
        (agg_oc,) = sc_agg(hr_oc, ridxP, dstP)
        aggs.append(agg_oc)

    aggp = jnp.stack(aggs, axis=1)
    b4 = b.reshape(OC, 1, LANES)

    out = pl.pallas_call(
        _combine_body,
        grid=(N_NODES // BN, OC),
        in_specs=[
            pl.BlockSpec((BN, IN_DIM), lambda i, oc: (i, 0)),
            pl.BlockSpec((IN_DIM, LANES), lambda i, oc: (0, oc)),
            pl.BlockSpec((NC, 1, BN, LANES), lambda i, oc: (0, oc, i, 0)),
            pl.BlockSpec((NC, BN, LANES), lambda i, oc: (0, i, 0)),
            pl.BlockSpec((1, 1, LANES), lambda i, oc: (oc, 0, 0)),
        ],
        out_specs=pl.BlockSpec((BN, LANES), lambda i, oc: (i, oc)),
        out_shape=jax.ShapeDtypeStruct((N_NODES, OUT_DIM), jnp.float32),
    )(x, W_self, aggp, degp, b4)
    return out


# speedup vs baseline: 1.4611x; 1.1244x over previous
import jax
import jax.numpy as jnp
from jax import lax
from jax.experimental import pallas as pl
from jax.experimental.pallas import tpu as pltpu
from jax.experimental.pallas import tpu_sc as plsc

N_NODES = 10000
N_EDGES = 160000
N_REL = 4
IN_DIM = 532
OUT_DIM = 512
OC = 4
LANES = 128
NC = 2
NS = 16
NW = NC * NS
EPW = N_EDGES // NW
BATCH = 128
NB = 40
PAD = NB * BATCH - EPW
TBL = N_REL * N_NODES
AGG_ROWS = 10112
ZSTRIPE = AGG_ROWS // NS
BN = 400


def _proj_body(x_ref, w_ref, o_ref):
    r = pl.program_id(1)
    o_ref[...] = jnp.dot(x_ref[...], w_ref[r],
                         preferred_element_type=jnp.float32)


def _combine_body(x_ref, ws_ref, a0, a1, a2, a3, deg_ref, b_ref, o_ref):
    self_term = jnp.dot(x_ref[...], ws_ref[...],
                        preferred_element_type=jnp.float32)
    agg = jnp.concatenate([a[0] + a[1] for a in (a0, a1, a2, a3)], axis=1)
    dg = deg_ref[0, :, :1] + deg_ref[1, :, :1]
    norm = 1.0 / jnp.maximum(dg, 1.0)
    o_ref[...] = jnp.maximum(self_term + agg * norm + b_ref[...], 0.0)


def _zero_stripe(zbuf, table, zrow):
    for z in range(19):
        pltpu.sync_copy(zbuf, table.at[pl.ds(zrow + z * 32, 32)])
    pltpu.sync_copy(zbuf.at[pl.ds(0, 24)], table.at[pl.ds(zrow + 608, 24)])


def _copyout_stripe(table, out, zrow):
    for qo, qn in ((0, 160), (160, 160), (320, 160), (480, 152)):
        pltpu.sync_copy(table.at[pl.ds(zrow + qo, qn)],
                        out.at[pl.ds(zrow + qo, qn)])


def _fill_zeros(buf, nrows):
    def _body(i, _):
        for k in range(LANES // 16):
            buf[i, pl.ds(k * 16, 16)] = jnp.zeros((16,), jnp.float32)
        return 0
    lax.fori_loop(0, nrows, _body, 0)


def _sc_agg_body(hr, ridxP, dstP, aggp, idxv, dstv, rows, zbuf,
                 aggsp, semA, semB):
    cid = lax.axis_index("c")
    sid = lax.axis_index("s")
    w = cid * NS + sid

    pltpu.sync_copy(ridxP.at[w], idxv)
    pltpu.sync_copy(dstP.at[w], dstv)
    _fill_zeros(zbuf, 32)

    zrow = sid * ZSTRIPE
    _zero_stripe(zbuf, aggsp, zrow)
    plsc.subcore_barrier()

    pltpu.async_copy(hr.at[idxv.at[0]], rows.at[0], semA)

    def _pair(p, _):
        j0 = 2 * p
        j1 = 2 * p + 1
        j2 = jnp.where(j1 + 1 < NB, j1 + 1, 0)
        pltpu.async_copy(hr.at[idxv.at[j1]], rows.at[1], semB)
        pltpu.make_async_copy(hr.at[idxv.at[j0]], rows.at[0], semA).wait()
        pltpu.sync_copy(rows.at[0], aggsp.at[dstv.at[j0]], add=True)
        pltpu.async_copy(hr.at[idxv.at[j2]], rows.at[0], semA)
        pltpu.make_async_copy(hr.at[idxv.at[j1]], rows.at[1], semB).wait()
        pltpu.sync_copy(rows.at[1], aggsp.at[dstv.at[j1]], add=True)
        return 0
    lax.fori_loop(0, NB // 2, _pair, 0)
    pltpu.make_async_copy(hr.at[idxv.at[0]], rows.at[0], semA).wait()
    plsc.subcore_barrier()

    _copyout_stripe(aggsp, aggp.at[cid], zrow)


def _sc_deg_body(dstP, degp, dstv, ones, zbuf, degsp):
    cid = lax.axis_index("c")
    sid = lax.axis_index("s")
    w = cid * NS + sid

    pltpu.sync_copy(dstP.at[w], dstv)
    _fill_zeros(zbuf, 32)

    def _fill_ones(i, _):
        for k in range(LANES // 16):
            ones[i, pl.ds(k * 16, 16)] = jnp.ones((16,), jnp.float32)
        return 0
    lax.fori_loop(0, BATCH, _fill_ones, 0)

    zrow = sid * ZSTRIPE
    _zero_stripe(zbuf, degsp, zrow)
    plsc.subcore_barrier()

    def _deg(j, _):
        pltpu.sync_copy(ones, degsp.at[dstv.at[j]], add=True)
        return 0
    lax.fori_loop(0, NB, _deg, 0)
    plsc.subcore_barrier()

    _copyout_stripe(degsp, degp.at[cid], zrow)


def kernel(x, edge_index, edge_type, W_rel, W_self, b):
    src = edge_index[0]
    dst = edge_index[1]

    W16 = W_rel.reshape(N_REL, IN_DIM, OC, LANES)
    W16 = W16.transpose(0, 2, 1, 3).reshape(N_REL * OC, IN_DIM, LANES)

    ridx = edge_type * N_NODES + src
    wi = jnp.arange(NW, dtype=jnp.int32)[:, None]
    ki = jnp.arange(PAD, dtype=jnp.int32)[None, :]
    pad_ridx = (wi * PAD + ki) % N_NODES
    pad_dst = N_NODES + (wi % NS) + jnp.zeros_like(ki)
    ridxP = jnp.concatenate([ridx.reshape(NW, EPW), pad_ridx], 1)
    ridxP = ridxP.reshape(NW, NB, BATCH)
    dstP = jnp.concatenate([dst.reshape(NW, EPW), pad_dst], 1)
    dstP = dstP.reshape(NW, NB, BATCH)

    mesh = plsc.VectorSubcoreMesh(core_axis_name="c", subcore_axis_name="s")

    sc_deg = pl.kernel(
        _sc_deg_body,
        out_type=[jax.ShapeDtypeStruct((NC, AGG_ROWS, LANES), jnp.float32)],
        mesh=mesh,
        scratch_types=[
            pltpu.VMEM((NB, BATCH), jnp.int32),
            pltpu.VMEM((BATCH, LANES), jnp.float32),
            pltpu.VMEM((32, LANES), jnp.float32),
            pltpu.VMEM_SHARED((AGG_ROWS, LANES), jnp.float32),
        ],
    )
    (degp,) = sc_deg(dstP)

    sc_agg = pl.kernel(
        _sc_agg_body,
        out_type=[jax.ShapeDtypeStruct((NC, AGG_ROWS, LANES), jnp.float32)],
        mesh=mesh,
        scratch_types=[
            pltpu.VMEM((NB, BATCH), jnp.int32),
            pltpu.VMEM((NB, BATCH), jnp.int32),
            pltpu.VMEM((2, BATCH, LANES), jnp.float32),
            pltpu.VMEM((32, LANES), jnp.float32),
            pltpu.VMEM_SHARED((AGG_ROWS, LANES), jnp.float32),
            pltpu.SemaphoreType.DMA,
            pltpu.SemaphoreType.DMA,
        ],
    )

    aggs = []
    for oc in range(OC):
        hr_oc = pl.pallas_call(
            _proj_body,
            grid=(N_NODES // BN, N_REL),
            in_specs=[
                pl.BlockSpec((BN, IN_DIM), lambda i, r: (i, 0)),
                pl.BlockSpec((N_REL, IN_DIM, LANES),
                             lambda i, r, _oc=oc: (0, 0, 0)),
            ],
            out_specs=pl.BlockSpec(
                (BN, LANES),
                lambda i, r: (r * (N_NODES // BN) + i, 0)),
            out_shape=jax.ShapeDtypeStruct((TBL, LANES), jnp.float32),
        )(x, lax.slice_in_dim(W16.reshape(N_REL, OC, IN_DIM, LANES),
                              oc, oc + 1, axis=1).reshape(N_REL, IN_DIM, LANES))
        (agg_oc,) = sc_agg(hr_oc, ridxP, dstP)
        aggs.append(agg_oc)

    b2 = b.reshape(1, OUT_DIM)

    agg_spec = pl.BlockSpec((NC, BN, LANES), lambda i: (0, i, 0))
    out = pl.pallas_call(
        _combine_body,
        grid=(N_NODES // BN,),
        in_specs=[
            pl.BlockSpec((BN, IN_DIM), lambda i: (i, 0)),
            pl.BlockSpec((IN_DIM, OUT_DIM), lambda i: (0, 0)),
            agg_spec, agg_spec, agg_spec, agg_spec,
            pl.BlockSpec((NC, BN, LANES), lambda i: (0, i, 0)),
            pl.BlockSpec((1, OUT_DIM), lambda i: (0, 0)),
        ],
        out_specs=pl.BlockSpec((BN, OUT_DIM), lambda i: (i, 0)),
        out_shape=jax.ShapeDtypeStruct((N_NODES, OUT_DIM), jnp.float32),
    )(x, W_self, aggs[0], aggs[1], aggs[2], aggs[3], degp, b2)
    return out
